# baseline (speedup 1.0000x reference)
import functools

import jax
import jax.numpy as jnp
from jax import lax
from jax.experimental import pallas as pl
from jax.experimental.pallas import tpu as pltpu

NCLS = 4
ALPHA = 0.8
LANES = 128
RB = 1024


def _body(cs_ref, gt_ref, ema_ref, out_ref, acc_ce, acc_h, *, grid):
    i = pl.program_id(0)

    @pl.when(i == 0)
    def _init():
        acc_ce[...] = jnp.zeros_like(acc_ce)
        acc_h[...] = jnp.zeros_like(acc_h)

    c0 = cs_ref[0].astype(jnp.float32)
    c1 = cs_ref[1].astype(jnp.float32)
    c2 = cs_ref[2].astype(jnp.float32)
    c3 = cs_ref[3].astype(jnp.float32)
    gt = gt_ref[...]

    m = jnp.maximum(jnp.maximum(c0, c1), jnp.maximum(c2, c3))
    e = (jnp.exp(c0 - m) + jnp.exp(c1 - m)
         + jnp.exp(c2 - m) + jnp.exp(c3 - m))
    lse = m + jnp.log(e)
    contrib = jnp.where(c0 == m, 1,
                        jnp.where(c1 == m, 1 << 8,
                                  jnp.where(c2 == m, 1 << 16, 1 << 24)))
    xg = jnp.where(gt == 0, c0,
                   jnp.where(gt == 1, c1,
                             jnp.where(gt == 2, c2, c3)))
    ce = lse - xg

    zf = jnp.zeros_like(ce)
    zi = jnp.zeros_like(contrib)
    for g in range(NCLS):
        og = gt == g
        acc_ce[g] += jnp.where(og, ce, zf)
        acc_h[g] += jnp.where(og, contrib, zi)

    @pl.when(i == grid - 1)
    def _epilogue():
        ema_v = ema_ref[...]
        kio = lax.broadcasted_iota(jnp.int32, (1, 16), 1)
        conf = [[jnp.sum(((acc_h[g] >> (8 * p)) & 255).astype(jnp.float32))
                 for p in range(NCLS)] for g in range(NCLS)]
        ema = [[ALPHA * conf[g][p]
                + (1.0 - ALPHA) * jnp.sum(
                    jnp.where(kio == NCLS * g + p, ema_v, 0.0))
                for p in range(NCLS)] for g in range(NCLS)]
        mispred = [sum(ema[g][p] for p in range(NCLS)) - ema[g][g]
                   for g in range(NCLS)]
        maxm = jnp.maximum(jnp.maximum(mispred[0], mispred[1]),
                           jnp.maximum(mispred[2], mispred[3]))
        loss = 0.0
        for g in range(NCLS):
            w = jnp.minimum(maxm / (mispred[g] + 1e-6), 1.2)
            s = jnp.sum(acc_ce[g])
            loss = loss + w * s
        out_ref[...] = jnp.broadcast_to(loss, (1, 1))


def kernel(lg, gt, ema_confusion):
    n = lg.shape[0]
    rows = n // LANES
    grid = rows // RB
    cs = jnp.transpose(lg.reshape(rows, LANES, NCLS).astype(jnp.bfloat16),
                       (2, 0, 1))
    gtr = gt.reshape(rows, LANES)
    ema16 = ema_confusion.reshape(1, 16)

    out = pl.pallas_call(
        functools.partial(_body, grid=grid),
        grid=(grid,),
        in_specs=[
            pl.BlockSpec((NCLS, RB, LANES), lambda i: (0, i, 0)),
            pl.BlockSpec((RB, LANES), lambda i: (i, 0)),
            pl.BlockSpec((1, 16), lambda i: (0, 0)),
        ],
        out_specs=pl.BlockSpec((1, 1), lambda i: (0, 0)),
        out_shape=jax.ShapeDtypeStruct((1, 1), jnp.float32),
        scratch_shapes=[
            pltpu.VMEM((NCLS, RB, LANES), jnp.float32),
            pltpu.VMEM((NCLS, RB, LANES), jnp.int32),
        ],
    )(cs, gtr, ema16)
    return jnp.reshape(out, ()) / n


# speedup vs baseline: 1.0262x; 1.0262x over previous
import functools

import jax
import jax.numpy as jnp
from jax import lax
from jax.experimental import pallas as pl
from jax.experimental.pallas import tpu as pltpu

NCLS = 4
ALPHA = 0.8
LANES = 128
RB = 1024


def _body(cs_ref, gt_ref, ema_ref, out_ref, acc_ce, acc_h, *, grid):
    i = pl.program_id(0)

    @pl.when(i == 0)
    def _init():
        acc_ce[...] = jnp.zeros_like(acc_ce)
        acc_h[...] = jnp.zeros_like(acc_h)

    c0 = cs_ref[0]
    c1 = cs_ref[1]
    c2 = cs_ref[2]
    c3 = cs_ref[3]
    gt = gt_ref[...]

    m = jnp.maximum(jnp.maximum(c0, c1), jnp.maximum(c2, c3))
    e = (jnp.exp(c0 - m) + jnp.exp(c1 - m)
         + jnp.exp(c2 - m) + jnp.exp(c3 - m))
    lse = m + jnp.log(e)
    contrib = jnp.where(c0 == m, 1,
                        jnp.where(c1 == m, 1 << 8,
                                  jnp.where(c2 == m, 1 << 16, 1 << 24)))
    xg = jnp.where(gt == 0, c0,
                   jnp.where(gt == 1, c1,
                             jnp.where(gt == 2, c2, c3)))
    ce = lse - xg

    zf = jnp.zeros_like(ce)
    zi = jnp.zeros_like(contrib)
    for g in range(NCLS):
        og = gt == g
        acc_ce[g] += jnp.where(og, ce, zf)
        acc_h[g] += jnp.where(og, contrib, zi)

    @pl.when(i == grid - 1)
    def _epilogue():
        ema_v = ema_ref[...]
        kio = lax.broadcasted_iota(jnp.int32, (1, 16), 1)
        conf = [[jnp.sum(((acc_h[g] >> (8 * p)) & 255).astype(jnp.float32))
                 for p in range(NCLS)] for g in range(NCLS)]
        ema = [[ALPHA * conf[g][p]
                + (1.0 - ALPHA) * jnp.sum(
                    jnp.where(kio == NCLS * g + p, ema_v, 0.0))
                for p in range(NCLS)] for g in range(NCLS)]
        mispred = [sum(ema[g][p] for p in range(NCLS)) - ema[g][g]
                   for g in range(NCLS)]
        maxm = jnp.maximum(jnp.maximum(mispred[0], mispred[1]),
                           jnp.maximum(mispred[2], mispred[3]))
        loss = 0.0
        for g in range(NCLS):
            w = jnp.minimum(maxm / (mispred[g] + 1e-6), 1.2)
            s = jnp.sum(acc_ce[g])
            loss = loss + w * s
        out_ref[...] = jnp.broadcast_to(loss, (1, 1))


def kernel(lg, gt, ema_confusion):
    n = lg.shape[0]
    rows = n // LANES
    grid = rows // RB
    cs = jnp.transpose(lg.reshape(rows, LANES, NCLS), (2, 0, 1))
    gtr = gt.reshape(rows, LANES)
    ema16 = ema_confusion.reshape(1, 16)

    out = pl.pallas_call(
        functools.partial(_body, grid=grid),
        grid=(grid,),
        in_specs=[
            pl.BlockSpec((NCLS, RB, LANES), lambda i: (0, i, 0)),
            pl.BlockSpec((RB, LANES), lambda i: (i, 0)),
            pl.BlockSpec((1, 16), lambda i: (0, 0)),
        ],
        out_specs=pl.BlockSpec((1, 1), lambda i: (0, 0)),
        out_shape=jax.ShapeDtypeStruct((1, 1), jnp.float32),
        scratch_shapes=[
            pltpu.VMEM((NCLS, RB, LANES), jnp.float32),
            pltpu.VMEM((NCLS, RB, LANES), jnp.int32),
        ],
    )(cs, gtr, ema16)
    return jnp.reshape(out, ()) / n
